# baseline (speedup 1.0000x reference)
import functools

import jax
import jax.numpy as jnp
from jax import lax
from jax.experimental import pallas as pl
from jax.experimental.pallas import tpu as pltpu
from jax.experimental.pallas import tpu_sc as plsc

WORLD_SIZE = 4
HALO = 8192
B = WORLD_SIZE * HALO
D = 256

_info = plsc.get_sparse_core_info()
NC = _info.num_cores
NS = _info.num_subcores
NW = NC * NS
B_PER_W = B // NW
CHUNK = 128
NCHUNK = B_PER_W // CHUNK
NBUF = 3


def _sc_gather(table, idx2d):
    mesh = plsc.VectorSubcoreMesh(core_axis_name="c", subcore_axis_name="s")

    @functools.partial(
        pl.kernel,
        mesh=mesh,
        out_type=jax.ShapeDtypeStruct((B, D), jnp.float32),
        scratch_types=(
            [pltpu.VMEM((NCHUNK, CHUNK), jnp.int32)]
            + [pltpu.VMEM((CHUNK, D), jnp.float32) for _ in range(NBUF)]
            + [pltpu.SemaphoreType.DMA for _ in range(2 * NBUF)]
        ),
    )
    def k(table_hbm, idx_hbm, out_hbm, idx_v, *bufs_and_sems):
        bufs = bufs_and_sems[:NBUF]
        gsem = bufs_and_sems[NBUF:2 * NBUF]
        osem = bufs_and_sems[2 * NBUF:]
        wid = lax.axis_index("s") * NC + lax.axis_index("c")
        base = wid * B_PER_W

        pltpu.sync_copy(idx_hbm.at[pl.ds(wid * NCHUNK, NCHUNK)], idx_v)

        def start_gather(j):
            return pltpu.async_copy(
                table_hbm.at[idx_v.at[j]], bufs[j % NBUF], gsem[j % NBUF])

        def start_out(j):
            return pltpu.async_copy(
                bufs[j % NBUF], out_hbm.at[pl.ds(base + j * CHUNK, CHUNK)],
                osem[j % NBUF])

        gathers = [None] * NCHUNK
        for j in range(min(NBUF - 1, NCHUNK)):
            gathers[j] = start_gather(j)
        outs = [None] * NCHUNK
        for j in range(NCHUNK):
            gathers[j].wait()
            outs[j] = start_out(j)
            nxt = j + NBUF - 1
            if nxt < NCHUNK:
                if j >= 1:
                    outs[j - 1].wait()
                gathers[nxt] = start_gather(nxt)
        for j in range(max(0, NCHUNK - NBUF + 1), NCHUNK):
            outs[j].wait()

    return k(table, idx2d)


def kernel(local, lidx):
    return _sc_gather(local, lidx.reshape(B // CHUNK, CHUNK))


# speedup vs baseline: 1.0271x; 1.0271x over previous
import functools

import jax
import jax.numpy as jnp
from jax import lax
from jax.experimental import pallas as pl
from jax.experimental.pallas import tpu as pltpu
from jax.experimental.pallas import tpu_sc as plsc

WORLD_SIZE = 4
HALO = 8192
B = WORLD_SIZE * HALO
D = 256

_info = plsc.get_sparse_core_info()
NC = _info.num_cores
NS = _info.num_subcores
NW = NC * NS
B_PER_W = B // NW
CHUNK = 64
NCHUNK = B_PER_W // CHUNK
NBUF = 6


def _sc_gather(table, idx2d):
    mesh = plsc.VectorSubcoreMesh(core_axis_name="c", subcore_axis_name="s")

    @functools.partial(
        pl.kernel,
        mesh=mesh,
        out_type=jax.ShapeDtypeStruct((B, D), jnp.float32),
        scratch_types=(
            [pltpu.VMEM((NCHUNK, CHUNK), jnp.int32)]
            + [pltpu.VMEM((CHUNK, D), jnp.float32) for _ in range(NBUF)]
            + [pltpu.SemaphoreType.DMA for _ in range(2 * NBUF)]
        ),
    )
    def k(table_hbm, idx_hbm, out_hbm, idx_v, *bufs_and_sems):
        bufs = bufs_and_sems[:NBUF]
        gsem = bufs_and_sems[NBUF:2 * NBUF]
        osem = bufs_and_sems[2 * NBUF:]
        wid = lax.axis_index("s") * NC + lax.axis_index("c")
        base = wid * B_PER_W

        pltpu.sync_copy(idx_hbm.at[pl.ds(wid * NCHUNK, NCHUNK)], idx_v)

        def start_gather(j):
            return pltpu.async_copy(
                table_hbm.at[idx_v.at[j]], bufs[j % NBUF], gsem[j % NBUF])

        def start_out(j):
            return pltpu.async_copy(
                bufs[j % NBUF], out_hbm.at[pl.ds(base + j * CHUNK, CHUNK)],
                osem[j % NBUF])

        gathers = [None] * NCHUNK
        for j in range(min(NBUF, NCHUNK)):
            gathers[j] = start_gather(j)
        outs = [None] * NCHUNK
        for j in range(NCHUNK):
            gathers[j].wait()
            outs[j] = start_out(j)
            if j + NBUF < NCHUNK:
                outs[j].wait()
                gathers[j + NBUF] = start_gather(j + NBUF)
        for j in range(max(0, NCHUNK - NBUF), NCHUNK):
            outs[j].wait()

    return k(table, idx2d)


def kernel(local, lidx):
    return _sc_gather(local, lidx.reshape(B // CHUNK, CHUNK))
